# Initial kernel scaffold; baseline (speedup 1.0000x reference)
import jax
import jax.numpy as jnp
from jax.experimental import pallas as pl


def kernel(x, edge_index, batch, W_emb, b_emb, lin1_W, lin1_b, bn_g, bn_b, lin2_W, lin2_b):
    raise NotImplementedError("write your pallas kernel here")


# speedup vs baseline: 7.8108x; 7.8108x over previous
import functools

import jax
import jax.numpy as jnp
from jax import lax
from jax.experimental import pallas as pl
from jax.experimental.pallas import tpu as pltpu
from jax.experimental.pallas import tpu_sc as plsc

N = 50000
E = 800000
D_IN = 128
H = 64
HH = 32
L = 4
G = 512

NC = 2
NS = 16
NW = NC * NS

CH = 80
KC = 5
SBE = 2000
EPT = E // NS
NSB = EPT // SBE
WPS = SBE // (CH * KC)
PCH = N // CH
PIT = -(-PCH // NW)
ZCH = 200
NZC = N // ZCH
ZIT = -(-NZC // NS)

_mesh = plsc.VectorSubcoreMesh(
    core_axis_name="c", subcore_axis_name="s", num_cores=NC, num_subcores=NS
)
_sc_params = pltpu.CompilerParams(
    use_tc_tiling_on_sc=False, needs_layout_passes=False
)


def _zero_vmem(ref, nrows, ncols):
    def body(i, _):
        for t in range(ncols // 16):
            ref[i, pl.ds(t * 16, 16)] = jnp.zeros((16,), jnp.float32)
        return 0

    lax.fori_loop(0, nrows, body, 0)


def _pool_loop(h2, batch1, pbuf, bbuf, bidx2, pacc2, w, ones=None, cacc2=None):
    iota = lax.iota(jnp.int32, 16)
    half = lax.shift_right_logical(iota, 1)
    par = lax.bitwise_and(iota, 1)

    def body(k, _):
        j = k * NW + w

        @pl.when(j < PCH)
        def _():
            pltpu.sync_copy(batch1.at[pl.ds(j * CH, CH)], bbuf)
            pltpu.sync_copy(h2.at[pl.ds(j * 2 * CH, 2 * CH)], pbuf)
            for t in range(2 * CH // 16):
                b16 = plsc.load_gather(bbuf, [half + t * 8])
                bidx2[t // 5, pl.ds((t % 5) * 16, 16)] = b16 + b16 + par
            for u in range(2):
                pltpu.sync_copy(
                    pbuf.at[pl.ds(u * CH, CH)], pacc2.at[bidx2.at[u]], add=True
                )
                if cacc2 is not None:
                    pltpu.sync_copy(ones, cacc2.at[bidx2.at[u]], add=True)

        return 0

    lax.fori_loop(0, PIT, body, 0)


def _agg_body(h2, src1, dst1, agg_out,
              srctmp, dsttmp, srcv, dstv, rows, zb, acc, sem):
    c = lax.axis_index("c")
    s = lax.axis_index("s")
    w = c * NS + s

    _zero_vmem(zb, ZCH, HH)

    def zcopy(k, _):
        j = k * NS + s

        @pl.when(j < NZC)
        def _():
            pltpu.sync_copy(zb, acc.at[pl.ds(j * ZCH, ZCH)])

        return 0

    lax.fori_loop(0, ZIT, zcopy, 0)
    plsc.subcore_barrier()

    def superblock(sb, _):
        e0 = s * EPT + sb * SBE
        pltpu.sync_copy(src1.at[pl.ds(e0, SBE)], srctmp)
        pltpu.sync_copy(dst1.at[pl.ds(e0, SBE)], dsttmp)
        for b in range(WPS):
            for k2 in range(KC):
                q = (b * KC + k2) * CH
                for t in range(CH // 16):
                    v = srctmp[pl.ds(q + t * 16, 16)]
                    srcv[k2, pl.ds(t * 16, 16)] = v + v + c
                    dstv[k2, pl.ds(t * 16, 16)] = dsttmp[pl.ds(q + t * 16, 16)]
            descs = []
            for k2 in range(KC):
                descs.append(
                    pltpu.async_copy(
                        h2.at[srcv.at[k2]], rows.at[pl.ds(k2 * CH, CH)], sem
                    )
                )
            for d in descs:
                d.wait()
            for k2 in range(KC):
                pltpu.sync_copy(
                    rows.at[pl.ds(k2 * CH, CH)], acc.at[dstv.at[k2]], add=True
                )
        return 0

    lax.fori_loop(0, NSB, superblock, 0)

    plsc.subcore_barrier()

    def wb(k, _):
        j = k * NS + s

        @pl.when(j < NZC)
        def _():
            pltpu.sync_copy(
                acc.at[pl.ds(j * ZCH, ZCH)], agg_out.at[c, pl.ds(j * ZCH, ZCH)]
            )

        return 0

    lax.fori_loop(0, ZIT, wb, 0)


_agg = functools.partial(
    pl.kernel,
    out_type=jax.ShapeDtypeStruct((NC, N, HH), jnp.float32),
    mesh=_mesh,
    scratch_types=[
        pltpu.VMEM((SBE,), jnp.int32),
        pltpu.VMEM((SBE,), jnp.int32),
        pltpu.VMEM((KC, CH), jnp.int32),
        pltpu.VMEM((KC, CH), jnp.int32),
        pltpu.VMEM((KC * CH, HH), jnp.float32),
        pltpu.VMEM((ZCH, HH), jnp.float32),
        pltpu.VMEM_SHARED((N, HH), jnp.float32),
        pltpu.SemaphoreType.DMA,
    ],
    compiler_params=_sc_params,
)(_agg_body)


GPT = 2 * G // NS


def _make_pool(with_cnt):
    def body(h2, batch1, *refs):
        if with_cnt:
            (pool_out, cnt_out, pbuf, bbuf, bidx2, ones, zb, pacc2, cacc2) = refs
        else:
            (pool_out, pbuf, bbuf, bidx2, zb, pacc2) = refs
            ones = cacc2 = cnt_out = None
        c = lax.axis_index("c")
        s = lax.axis_index("s")
        w = c * NS + s

        _zero_vmem(zb, GPT, HH)
        pltpu.sync_copy(zb, pacc2.at[pl.ds(s * GPT, GPT)])
        if with_cnt:
            pltpu.sync_copy(zb, cacc2.at[pl.ds(s * GPT, GPT)])

            def fill1(i, _):
                for t in range(HH // 16):
                    ones[i, pl.ds(t * 16, 16)] = jnp.ones((16,), jnp.float32)
                return 0

            lax.fori_loop(0, CH, fill1, 0)
        plsc.subcore_barrier()

        _pool_loop(h2, batch1, pbuf, bbuf, bidx2, pacc2, w, ones=ones, cacc2=cacc2)

        plsc.subcore_barrier()
        pltpu.sync_copy(pacc2.at[pl.ds(s * GPT, GPT)], pool_out.at[c, pl.ds(s * GPT, GPT)])
        if with_cnt:
            pltpu.sync_copy(cacc2.at[pl.ds(s * GPT, GPT)], cnt_out.at[c, pl.ds(s * GPT, GPT)])

    shp = jax.ShapeDtypeStruct((NC, 2 * G, HH), jnp.float32)
    scratch = [
        pltpu.VMEM((2 * CH, HH), jnp.float32),
        pltpu.VMEM((CH,), jnp.int32),
        pltpu.VMEM((2, CH), jnp.int32),
    ]
    if with_cnt:
        scratch.append(pltpu.VMEM((CH, HH), jnp.float32))
    scratch.append(pltpu.VMEM((GPT, HH), jnp.float32))
    scratch.append(pltpu.VMEM_SHARED((2 * G, HH), jnp.float32))
    if with_cnt:
        scratch.append(pltpu.VMEM_SHARED((2 * G, HH), jnp.float32))
    return functools.partial(
        pl.kernel,
        out_type=(shp, shp) if with_cnt else shp,
        mesh=_mesh,
        scratch_types=scratch,
        compiler_params=_sc_params,
    )(body)


_pool = _make_pool(False)
_pool_cnt = _make_pool(True)


T = 1000
NT = N // T


def _embed_body(x_ref, w_ref, b_ref, o_ref):
    o_ref[...] = (
        jnp.dot(x_ref[...], w_ref[...], preferred_element_type=jnp.float32)
        + b_ref[...]
    )


_embed = pl.pallas_call(
    _embed_body,
    grid=(NT,),
    in_specs=[
        pl.BlockSpec((T, D_IN), lambda i: (i, 0)),
        pl.BlockSpec((D_IN, H), lambda i: (0, 0)),
        pl.BlockSpec((1, H), lambda i: (0, 0)),
    ],
    out_specs=pl.BlockSpec((T, H), lambda i: (i, 0)),
    out_shape=jax.ShapeDtypeStruct((N, H), jnp.float32),
)


def _l1_body(h_ref, a_ref, w_ref, b_ref, z_ref, s_ref, ss_ref):
    av = jnp.concatenate([a_ref[0], a_ref[1]], axis=1)
    z = (
        jnp.dot(h_ref[...] + av, w_ref[...], preferred_element_type=jnp.float32)
        + b_ref[...]
    )
    z_ref[...] = z

    @pl.when(pl.program_id(0) == 0)
    def _():
        s_ref[...] = jnp.zeros_like(s_ref)
        ss_ref[...] = jnp.zeros_like(ss_ref)

    s_ref[...] += jnp.sum(z, axis=0, keepdims=True)
    ss_ref[...] += jnp.sum(z * z, axis=0, keepdims=True)


_l1 = pl.pallas_call(
    _l1_body,
    grid=(NT,),
    in_specs=[
        pl.BlockSpec((T, H), lambda i: (i, 0)),
        pl.BlockSpec((NC, T, HH), lambda i: (0, i, 0)),
        pl.BlockSpec((H, H), lambda i: (0, 0)),
        pl.BlockSpec((1, H), lambda i: (0, 0)),
    ],
    out_specs=[
        pl.BlockSpec((T, H), lambda i: (i, 0)),
        pl.BlockSpec((1, H), lambda i: (0, 0)),
        pl.BlockSpec((1, H), lambda i: (0, 0)),
    ],
    out_shape=[
        jax.ShapeDtypeStruct((N, H), jnp.float32),
        jax.ShapeDtypeStruct((1, H), jnp.float32),
        jax.ShapeDtypeStruct((1, H), jnp.float32),
    ],
)


def _l2_body(z_ref, s_ref, ss_ref, g_ref, bb_ref, w_ref, b2_ref, o_ref):
    mu = s_ref[...] * (1.0 / N)
    var = ss_ref[...] * (1.0 / N) - mu * mu
    zn = (z_ref[...] - mu) * lax.rsqrt(var + 1e-5) * g_ref[...] + bb_ref[...]
    zr = jnp.maximum(zn, 0.0)
    o_ref[...] = (
        jnp.dot(zr, w_ref[...], preferred_element_type=jnp.float32) + b2_ref[...]
    )


_l2 = pl.pallas_call(
    _l2_body,
    grid=(NT,),
    in_specs=[
        pl.BlockSpec((T, H), lambda i: (i, 0)),
        pl.BlockSpec((1, H), lambda i: (0, 0)),
        pl.BlockSpec((1, H), lambda i: (0, 0)),
        pl.BlockSpec((1, H), lambda i: (0, 0)),
        pl.BlockSpec((1, H), lambda i: (0, 0)),
        pl.BlockSpec((H, H), lambda i: (0, 0)),
        pl.BlockSpec((1, H), lambda i: (0, 0)),
    ],
    out_specs=pl.BlockSpec((T, H), lambda i: (i, 0)),
    out_shape=jax.ShapeDtypeStruct((N, H), jnp.float32),
)


def _fin_body(c_ref, p0_ref, p1_ref, p2_ref, p3_ref, p4_ref, o_ref):
    inv = 1.0 / jnp.maximum(jnp.sum(c_ref[...], axis=0), 1.0)
    parts = [
        jnp.sum(p_ref[...], axis=0) * inv
        for p_ref in (p0_ref, p1_ref, p2_ref, p3_ref, p4_ref)
    ]
    o_ref[...] = jnp.concatenate(parts, axis=1)


_fin = pl.pallas_call(
    _fin_body,
    out_shape=jax.ShapeDtypeStruct((G, (L + 1) * H), jnp.float32),
)


def kernel(x, edge_index, batch, W_emb, b_emb, lin1_W, lin1_b, bn_g, bn_b, lin2_W, lin2_b):
    src1 = edge_index[0]
    dst1 = edge_index[1]

    h = _embed(x, W_emb.T, b_emb.reshape(1, H))
    pools = []
    for l in range(L):
        h2 = h.reshape(NC * N, HH)
        agg = _agg(h2, src1, dst1)
        pools.append(_pool(h2, batch).reshape(NC, G, H))
        z1, zs, zss = _l1(h, agg, lin1_W[l].T, lin1_b[l].reshape(1, H))
        h = _l2(z1, zs, zss, bn_g[l].reshape(1, H), bn_b[l].reshape(1, H),
                lin2_W[l].T, lin2_b[l].reshape(1, H))
    pool_last, cnt = _pool_cnt(h.reshape(NC * N, HH), batch)
    pools.append(pool_last.reshape(NC, G, H))
    return _fin(cnt.reshape(NC, G, H), *pools)


# speedup vs baseline: 9.4900x; 1.2150x over previous
import functools

import jax
import jax.numpy as jnp
from jax import lax
from jax.experimental import pallas as pl
from jax.experimental.pallas import tpu as pltpu
from jax.experimental.pallas import tpu_sc as plsc

N = 50000
E = 800000
D_IN = 128
H = 64
HH = 32
L = 4
G = 512

NC = 2
NS = 16
NW = NC * NS

CH = 80
KC = 5
GRP = CH * KC
EPT = E // NS
NGW = EPT // GRP
PCH = N // CH
PIT = -(-PCH // NW)
ZCH = 80
NZC = N // ZCH
ZIT = -(-NZC // NS)

_mesh = plsc.VectorSubcoreMesh(
    core_axis_name="c", subcore_axis_name="s", num_cores=NC, num_subcores=NS
)
_sc_params = pltpu.CompilerParams(
    use_tc_tiling_on_sc=False, needs_layout_passes=False
)


def _zero_vmem(ref, nrows, ncols):
    def body(i, _):
        for t in range(ncols // 16):
            ref[i, pl.ds(t * 16, 16)] = jnp.zeros((16,), jnp.float32)
        return 0

    lax.fori_loop(0, nrows, body, 0)


def _pool_loop(h2, batch1, pbuf, bbuf, bidx2, pacc2, w, ones=None, cacc2=None):
    iota = lax.iota(jnp.int32, 16)
    half = lax.shift_right_logical(iota, 1)
    par = lax.bitwise_and(iota, 1)

    def body(k, _):
        j = k * NW + w

        @pl.when(j < PCH)
        def _():
            pltpu.sync_copy(batch1.at[pl.ds(j * CH, CH)], bbuf)
            pltpu.sync_copy(h2.at[pl.ds(j * 2 * CH, 2 * CH)], pbuf)
            for t in range(2 * CH // 16):
                b16 = plsc.load_gather(bbuf, [half + t * 8])
                bidx2[t // 5, pl.ds((t % 5) * 16, 16)] = b16 + b16 + par
            for u in range(2):
                pltpu.sync_copy(
                    pbuf.at[pl.ds(u * CH, CH)], pacc2.at[bidx2.at[u]], add=True
                )
                if cacc2 is not None:
                    pltpu.sync_copy(ones, cacc2.at[bidx2.at[u]], add=True)

        return 0

    lax.fori_loop(0, PIT, body, 0)


def _agg_body(h2, srcR, dstR, agg_out,
              srcv, dstv, rows, zb, acc, si0, si1, sg0, sg1):
    c = lax.axis_index("c")
    s = lax.axis_index("s")
    w = c * NS + s
    r_base = s * (EPT // CH)

    _zero_vmem(zb, ZCH, HH)

    def zcopy(k, _):
        j = k * NS + s

        @pl.when(j < NZC)
        def _():
            pltpu.sync_copy(zb, acc.at[pl.ds(j * ZCH, ZCH)])

        return 0

    lax.fori_loop(0, ZIT, zcopy, 0)
    plsc.subcore_barrier()

    sis = (si0, si1)
    sgs = (sg0, sg1)

    def fire_idx(b, g):
        pltpu.async_copy(srcR.at[pl.ds(r_base + g * KC, KC)], srcv.at[b], sis[b])
        pltpu.async_copy(dstR.at[pl.ds(r_base + g * KC, KC)], dstv.at[b], sis[b])

    def wait_idx(b):
        pltpu.make_async_copy(srcR.at[pl.ds(0, KC)], srcv.at[b], sis[b]).wait()
        pltpu.make_async_copy(dstR.at[pl.ds(0, KC)], dstv.at[b], sis[b]).wait()

    def fire_gathers(b):
        for k2 in range(KC):
            for t in range(CH // 16):
                v = srcv[b, k2, pl.ds(t * 16, 16)]
                srcv[b, k2, pl.ds(t * 16, 16)] = v + v + c
        for k2 in range(KC):
            pltpu.async_copy(
                h2.at[srcv.at[b, k2]], rows.at[b, pl.ds(k2 * CH, CH)], sgs[b]
            )

    def drain_scatter(b):
        pltpu.make_async_copy(h2.at[pl.ds(0, GRP)], rows.at[b], sgs[b]).wait()
        for k2 in range(KC):
            pltpu.sync_copy(
                rows.at[b, pl.ds(k2 * CH, CH)], acc.at[dstv.at[b, k2]], add=True
            )

    fire_idx(0, 0)
    wait_idx(0)
    fire_gathers(0)
    fire_idx(1, 1)

    def pipelined(k, _):
        g = 2 * k
        wait_idx(1)
        fire_gathers(1)
        drain_scatter(0)
        fire_idx(0, g + 2)
        wait_idx(0)
        fire_gathers(0)
        drain_scatter(1)

        @pl.when(g + 3 < NGW)
        def _():
            fire_idx(1, g + 3)

        return 0

    lax.fori_loop(0, (NGW - 1) // 2, pipelined, 0)
    drain_scatter(0)

    plsc.subcore_barrier()

    def wb(k, _):
        j = k * NS + s

        @pl.when(j < NZC)
        def _():
            pltpu.sync_copy(
                acc.at[pl.ds(j * ZCH, ZCH)], agg_out.at[c, pl.ds(j * ZCH, ZCH)]
            )

        return 0

    lax.fori_loop(0, ZIT, wb, 0)


_agg = functools.partial(
    pl.kernel,
    out_type=jax.ShapeDtypeStruct((NC, N, HH), jnp.float32),
    mesh=_mesh,
    scratch_types=[
        pltpu.VMEM((2, KC, CH), jnp.int32),
        pltpu.VMEM((2, KC, CH), jnp.int32),
        pltpu.VMEM((2, GRP, HH), jnp.float32),
        pltpu.VMEM((ZCH, HH), jnp.float32),
        pltpu.VMEM_SHARED((N, HH), jnp.float32),
        pltpu.SemaphoreType.DMA,
        pltpu.SemaphoreType.DMA,
        pltpu.SemaphoreType.DMA,
        pltpu.SemaphoreType.DMA,
    ],
    compiler_params=_sc_params,
)(_agg_body)


GPT = 2 * G // NS


def _make_pool(with_cnt):
    def body(h2, batch1, *refs):
        if with_cnt:
            (pool_out, cnt_out, pbuf, bbuf, bidx2, ones, zb, pacc2, cacc2) = refs
        else:
            (pool_out, pbuf, bbuf, bidx2, zb, pacc2) = refs
            ones = cacc2 = cnt_out = None
        c = lax.axis_index("c")
        s = lax.axis_index("s")
        w = c * NS + s

        _zero_vmem(zb, GPT, HH)
        pltpu.sync_copy(zb, pacc2.at[pl.ds(s * GPT, GPT)])
        if with_cnt:
            pltpu.sync_copy(zb, cacc2.at[pl.ds(s * GPT, GPT)])

            def fill1(i, _):
                for t in range(HH // 16):
                    ones[i, pl.ds(t * 16, 16)] = jnp.ones((16,), jnp.float32)
                return 0

            lax.fori_loop(0, CH, fill1, 0)
        plsc.subcore_barrier()

        _pool_loop(h2, batch1, pbuf, bbuf, bidx2, pacc2, w, ones=ones, cacc2=cacc2)

        plsc.subcore_barrier()
        pltpu.sync_copy(pacc2.at[pl.ds(s * GPT, GPT)], pool_out.at[c, pl.ds(s * GPT, GPT)])
        if with_cnt:
            pltpu.sync_copy(cacc2.at[pl.ds(s * GPT, GPT)], cnt_out.at[c, pl.ds(s * GPT, GPT)])

    shp = jax.ShapeDtypeStruct((NC, 2 * G, HH), jnp.float32)
    scratch = [
        pltpu.VMEM((2 * CH, HH), jnp.float32),
        pltpu.VMEM((CH,), jnp.int32),
        pltpu.VMEM((2, CH), jnp.int32),
    ]
    if with_cnt:
        scratch.append(pltpu.VMEM((CH, HH), jnp.float32))
    scratch.append(pltpu.VMEM((GPT, HH), jnp.float32))
    scratch.append(pltpu.VMEM_SHARED((2 * G, HH), jnp.float32))
    if with_cnt:
        scratch.append(pltpu.VMEM_SHARED((2 * G, HH), jnp.float32))
    return functools.partial(
        pl.kernel,
        out_type=(shp, shp) if with_cnt else shp,
        mesh=_mesh,
        scratch_types=scratch,
        compiler_params=_sc_params,
    )(body)


_pool = _make_pool(False)
_pool_cnt = _make_pool(True)


T = 1000
NT = N // T


def _embed_body(x_ref, w_ref, b_ref, o_ref):
    o_ref[...] = (
        jnp.dot(x_ref[...], w_ref[...], preferred_element_type=jnp.float32)
        + b_ref[...]
    )


_embed = pl.pallas_call(
    _embed_body,
    grid=(NT,),
    in_specs=[
        pl.BlockSpec((T, D_IN), lambda i: (i, 0)),
        pl.BlockSpec((D_IN, H), lambda i: (0, 0)),
        pl.BlockSpec((1, H), lambda i: (0, 0)),
    ],
    out_specs=pl.BlockSpec((T, H), lambda i: (i, 0)),
    out_shape=jax.ShapeDtypeStruct((N, H), jnp.float32),
)


def _l1_body(h_ref, a_ref, w_ref, b_ref, z_ref, s_ref, ss_ref):
    av = jnp.concatenate([a_ref[0], a_ref[1]], axis=1)
    z = (
        jnp.dot(h_ref[...] + av, w_ref[...], preferred_element_type=jnp.float32)
        + b_ref[...]
    )
    z_ref[...] = z

    @pl.when(pl.program_id(0) == 0)
    def _():
        s_ref[...] = jnp.zeros_like(s_ref)
        ss_ref[...] = jnp.zeros_like(ss_ref)

    s_ref[...] += jnp.sum(z, axis=0, keepdims=True)
    ss_ref[...] += jnp.sum(z * z, axis=0, keepdims=True)


_l1 = pl.pallas_call(
    _l1_body,
    grid=(NT,),
    in_specs=[
        pl.BlockSpec((T, H), lambda i: (i, 0)),
        pl.BlockSpec((NC, T, HH), lambda i: (0, i, 0)),
        pl.BlockSpec((H, H), lambda i: (0, 0)),
        pl.BlockSpec((1, H), lambda i: (0, 0)),
    ],
    out_specs=[
        pl.BlockSpec((T, H), lambda i: (i, 0)),
        pl.BlockSpec((1, H), lambda i: (0, 0)),
        pl.BlockSpec((1, H), lambda i: (0, 0)),
    ],
    out_shape=[
        jax.ShapeDtypeStruct((N, H), jnp.float32),
        jax.ShapeDtypeStruct((1, H), jnp.float32),
        jax.ShapeDtypeStruct((1, H), jnp.float32),
    ],
)


def _l2_body(z_ref, s_ref, ss_ref, g_ref, bb_ref, w_ref, b2_ref, o_ref):
    mu = s_ref[...] * (1.0 / N)
    var = ss_ref[...] * (1.0 / N) - mu * mu
    zn = (z_ref[...] - mu) * lax.rsqrt(var + 1e-5) * g_ref[...] + bb_ref[...]
    zr = jnp.maximum(zn, 0.0)
    o_ref[...] = (
        jnp.dot(zr, w_ref[...], preferred_element_type=jnp.float32) + b2_ref[...]
    )


_l2 = pl.pallas_call(
    _l2_body,
    grid=(NT,),
    in_specs=[
        pl.BlockSpec((T, H), lambda i: (i, 0)),
        pl.BlockSpec((1, H), lambda i: (0, 0)),
        pl.BlockSpec((1, H), lambda i: (0, 0)),
        pl.BlockSpec((1, H), lambda i: (0, 0)),
        pl.BlockSpec((1, H), lambda i: (0, 0)),
        pl.BlockSpec((H, H), lambda i: (0, 0)),
        pl.BlockSpec((1, H), lambda i: (0, 0)),
    ],
    out_specs=pl.BlockSpec((T, H), lambda i: (i, 0)),
    out_shape=jax.ShapeDtypeStruct((N, H), jnp.float32),
)


def _fin_body(c_ref, p0_ref, p1_ref, p2_ref, p3_ref, p4_ref, o_ref):
    inv = 1.0 / jnp.maximum(jnp.sum(c_ref[...], axis=0), 1.0)
    parts = [
        jnp.sum(p_ref[...], axis=0) * inv
        for p_ref in (p0_ref, p1_ref, p2_ref, p3_ref, p4_ref)
    ]
    o_ref[...] = jnp.concatenate(parts, axis=1)


_fin = pl.pallas_call(
    _fin_body,
    out_shape=jax.ShapeDtypeStruct((G, (L + 1) * H), jnp.float32),
)


def kernel(x, edge_index, batch, W_emb, b_emb, lin1_W, lin1_b, bn_g, bn_b, lin2_W, lin2_b):
    srcR = edge_index[0].reshape(E // CH, CH)
    dstR = edge_index[1].reshape(E // CH, CH)

    h = _embed(x, W_emb.T, b_emb.reshape(1, H))
    pools = []
    for l in range(L):
        h2 = h.reshape(NC * N, HH)
        agg = _agg(h2, srcR, dstR)
        pools.append(_pool(h2, batch).reshape(NC, G, H))
        z1, zs, zss = _l1(h, agg, lin1_W[l].T, lin1_b[l].reshape(1, H))
        h = _l2(z1, zs, zss, bn_g[l].reshape(1, H), bn_b[l].reshape(1, H),
                lin2_W[l].T, lin2_b[l].reshape(1, H))
    pool_last, cnt = _pool_cnt(h.reshape(NC * N, HH), batch)
    pools.append(pool_last.reshape(NC, G, H))
    return _fin(cnt.reshape(NC, G, H), *pools)


# speedup vs baseline: 10.0680x; 1.0609x over previous
import functools

import jax
import jax.numpy as jnp
from jax import lax
from jax.experimental import pallas as pl
from jax.experimental.pallas import tpu as pltpu
from jax.experimental.pallas import tpu_sc as plsc

N = 50000
E = 800000
D_IN = 128
H = 64
HH = 32
L = 4
G = 512

NC = 2
NS = 16
NW = NC * NS

CH = 80
KC = 5
GRP = CH * KC
EPT = E // NS
NGW = EPT // GRP
PCH = N // CH
PIT = -(-PCH // NW)
ZCH = 80
NZC = N // ZCH
ZIT = -(-NZC // NS)

_mesh = plsc.VectorSubcoreMesh(
    core_axis_name="c", subcore_axis_name="s", num_cores=NC, num_subcores=NS
)
_sc_params = pltpu.CompilerParams(
    use_tc_tiling_on_sc=False, needs_layout_passes=False
)


def _zero_vmem(ref, nrows, ncols):
    def body(i, _):
        for t in range(ncols // 16):
            ref[i, pl.ds(t * 16, 16)] = jnp.zeros((16,), jnp.float32)
        return 0

    lax.fori_loop(0, nrows, body, 0)


def _pool_loop(h2, batch1, pbuf, bbuf, bidx2, pacc2, w, ones=None, cacc2=None):
    iota = lax.iota(jnp.int32, 16)
    half = lax.shift_right_logical(iota, 1)
    par = lax.bitwise_and(iota, 1)

    def body(k, _):
        j = k * NW + w

        @pl.when(j < PCH)
        def _():
            pltpu.sync_copy(batch1.at[pl.ds(j * CH, CH)], bbuf)
            pltpu.sync_copy(h2.at[pl.ds(j * 2 * CH, 2 * CH)], pbuf)
            for t in range(2 * CH // 16):
                b16 = plsc.load_gather(bbuf, [half + t * 8])
                bidx2[t // 5, pl.ds((t % 5) * 16, 16)] = b16 + b16 + par
            for u in range(2):
                pltpu.sync_copy(
                    pbuf.at[pl.ds(u * CH, CH)], pacc2.at[bidx2.at[u]], add=True
                )
                if cacc2 is not None:
                    pltpu.sync_copy(ones, cacc2.at[bidx2.at[u]], add=True)

        return 0

    lax.fori_loop(0, PIT, body, 0)


def _agg_body(h2, srcR, dstR, agg_out,
              srcv, dstv, rows, zb, acc, si0, si1, sg0, sg1):
    c = lax.axis_index("c")
    s = lax.axis_index("s")
    w = c * NS + s
    r_base = s * (EPT // CH)

    _zero_vmem(zb, ZCH, HH)

    def zcopy(k, _):
        j = k * NS + s

        @pl.when(j < NZC)
        def _():
            pltpu.sync_copy(zb, acc.at[pl.ds(j * ZCH, ZCH)])

        return 0

    lax.fori_loop(0, ZIT, zcopy, 0)
    plsc.subcore_barrier()

    sis = (si0, si1)
    sgs = (sg0, sg1)

    def fire_idx(b, g):
        pltpu.async_copy(srcR.at[pl.ds(r_base + g * KC, KC)], srcv.at[b], sis[b])
        pltpu.async_copy(dstR.at[pl.ds(r_base + g * KC, KC)], dstv.at[b], sis[b])

    def wait_idx(b):
        pltpu.make_async_copy(srcR.at[pl.ds(0, KC)], srcv.at[b], sis[b]).wait()
        pltpu.make_async_copy(dstR.at[pl.ds(0, KC)], dstv.at[b], sis[b]).wait()

    def fire_gathers(b):
        for k2 in range(KC):
            for t in range(CH // 16):
                v = srcv[b, k2, pl.ds(t * 16, 16)]
                srcv[b, k2, pl.ds(t * 16, 16)] = v + v + c
        for k2 in range(KC):
            pltpu.async_copy(
                h2.at[srcv.at[b, k2]], rows.at[b, pl.ds(k2 * CH, CH)], sgs[b]
            )

    def drain_scatter(b):
        pltpu.make_async_copy(h2.at[pl.ds(0, GRP)], rows.at[b], sgs[b]).wait()
        for k2 in range(KC):
            pltpu.async_copy(
                rows.at[b, pl.ds(k2 * CH, CH)], acc.at[dstv.at[b, k2]], sis[b],
                add=True,
            )
        pltpu.make_async_copy(rows.at[b], acc.at[pl.ds(0, GRP)], sis[b]).wait()

    fire_idx(0, 0)
    wait_idx(0)
    fire_gathers(0)
    fire_idx(1, 1)

    def pipelined(k, _):
        g = 2 * k
        wait_idx(1)
        fire_gathers(1)
        drain_scatter(0)
        fire_idx(0, g + 2)
        wait_idx(0)
        fire_gathers(0)
        drain_scatter(1)

        @pl.when(g + 3 < NGW)
        def _():
            fire_idx(1, g + 3)

        return 0

    lax.fori_loop(0, (NGW - 1) // 2, pipelined, 0)
    drain_scatter(0)

    plsc.subcore_barrier()

    def wb(k, _):
        j = k * NS + s

        @pl.when(j < NZC)
        def _():
            pltpu.sync_copy(
                acc.at[pl.ds(j * ZCH, ZCH)], agg_out.at[c, pl.ds(j * ZCH, ZCH)]
            )

        return 0

    lax.fori_loop(0, ZIT, wb, 0)


_agg = functools.partial(
    pl.kernel,
    out_type=jax.ShapeDtypeStruct((NC, N, HH), jnp.float32),
    mesh=_mesh,
    scratch_types=[
        pltpu.VMEM((2, KC, CH), jnp.int32),
        pltpu.VMEM((2, KC, CH), jnp.int32),
        pltpu.VMEM((2, GRP, HH), jnp.float32),
        pltpu.VMEM((ZCH, HH), jnp.float32),
        pltpu.VMEM_SHARED((N, HH), jnp.float32),
        pltpu.SemaphoreType.DMA,
        pltpu.SemaphoreType.DMA,
        pltpu.SemaphoreType.DMA,
        pltpu.SemaphoreType.DMA,
    ],
    compiler_params=_sc_params,
)(_agg_body)


GPT = 2 * G // NS


def _make_pool(with_cnt):
    def body(h2, batch1, *refs):
        if with_cnt:
            (pool_out, cnt_out, pbuf, bbuf, bidx2, ones, zb, pacc2, cacc2) = refs
        else:
            (pool_out, pbuf, bbuf, bidx2, zb, pacc2) = refs
            ones = cacc2 = cnt_out = None
        c = lax.axis_index("c")
        s = lax.axis_index("s")
        w = c * NS + s

        _zero_vmem(zb, GPT, HH)
        pltpu.sync_copy(zb, pacc2.at[pl.ds(s * GPT, GPT)])
        if with_cnt:
            pltpu.sync_copy(zb, cacc2.at[pl.ds(s * GPT, GPT)])

            def fill1(i, _):
                for t in range(HH // 16):
                    ones[i, pl.ds(t * 16, 16)] = jnp.ones((16,), jnp.float32)
                return 0

            lax.fori_loop(0, CH, fill1, 0)
        plsc.subcore_barrier()

        _pool_loop(h2, batch1, pbuf, bbuf, bidx2, pacc2, w, ones=ones, cacc2=cacc2)

        plsc.subcore_barrier()
        pltpu.sync_copy(pacc2.at[pl.ds(s * GPT, GPT)], pool_out.at[c, pl.ds(s * GPT, GPT)])
        if with_cnt:
            pltpu.sync_copy(cacc2.at[pl.ds(s * GPT, GPT)], cnt_out.at[c, pl.ds(s * GPT, GPT)])

    shp = jax.ShapeDtypeStruct((NC, 2 * G, HH), jnp.float32)
    scratch = [
        pltpu.VMEM((2 * CH, HH), jnp.float32),
        pltpu.VMEM((CH,), jnp.int32),
        pltpu.VMEM((2, CH), jnp.int32),
    ]
    if with_cnt:
        scratch.append(pltpu.VMEM((CH, HH), jnp.float32))
    scratch.append(pltpu.VMEM((GPT, HH), jnp.float32))
    scratch.append(pltpu.VMEM_SHARED((2 * G, HH), jnp.float32))
    if with_cnt:
        scratch.append(pltpu.VMEM_SHARED((2 * G, HH), jnp.float32))
    return functools.partial(
        pl.kernel,
        out_type=(shp, shp) if with_cnt else shp,
        mesh=_mesh,
        scratch_types=scratch,
        compiler_params=_sc_params,
    )(body)


_pool = _make_pool(False)
_pool_cnt = _make_pool(True)


T = 1000
NT = N // T


def _embed_body(x_ref, w_ref, b_ref, o_ref):
    o_ref[...] = (
        jnp.dot(x_ref[...], w_ref[...], preferred_element_type=jnp.float32)
        + b_ref[...]
    )


_embed = pl.pallas_call(
    _embed_body,
    grid=(NT,),
    in_specs=[
        pl.BlockSpec((T, D_IN), lambda i: (i, 0)),
        pl.BlockSpec((D_IN, H), lambda i: (0, 0)),
        pl.BlockSpec((1, H), lambda i: (0, 0)),
    ],
    out_specs=pl.BlockSpec((T, H), lambda i: (i, 0)),
    out_shape=jax.ShapeDtypeStruct((N, H), jnp.float32),
)


def _l1_body(h_ref, a_ref, w_ref, b_ref, z_ref, s_ref, ss_ref):
    av = jnp.concatenate([a_ref[0], a_ref[1]], axis=1)
    z = (
        jnp.dot(h_ref[...] + av, w_ref[...], preferred_element_type=jnp.float32)
        + b_ref[...]
    )
    z_ref[...] = z

    @pl.when(pl.program_id(0) == 0)
    def _():
        s_ref[...] = jnp.zeros_like(s_ref)
        ss_ref[...] = jnp.zeros_like(ss_ref)

    s_ref[...] += jnp.sum(z, axis=0, keepdims=True)
    ss_ref[...] += jnp.sum(z * z, axis=0, keepdims=True)


_l1 = pl.pallas_call(
    _l1_body,
    grid=(NT,),
    in_specs=[
        pl.BlockSpec((T, H), lambda i: (i, 0)),
        pl.BlockSpec((NC, T, HH), lambda i: (0, i, 0)),
        pl.BlockSpec((H, H), lambda i: (0, 0)),
        pl.BlockSpec((1, H), lambda i: (0, 0)),
    ],
    out_specs=[
        pl.BlockSpec((T, H), lambda i: (i, 0)),
        pl.BlockSpec((1, H), lambda i: (0, 0)),
        pl.BlockSpec((1, H), lambda i: (0, 0)),
    ],
    out_shape=[
        jax.ShapeDtypeStruct((N, H), jnp.float32),
        jax.ShapeDtypeStruct((1, H), jnp.float32),
        jax.ShapeDtypeStruct((1, H), jnp.float32),
    ],
)


def _l2_body(z_ref, s_ref, ss_ref, g_ref, bb_ref, w_ref, b2_ref, o_ref):
    mu = s_ref[...] * (1.0 / N)
    var = ss_ref[...] * (1.0 / N) - mu * mu
    zn = (z_ref[...] - mu) * lax.rsqrt(var + 1e-5) * g_ref[...] + bb_ref[...]
    zr = jnp.maximum(zn, 0.0)
    o_ref[...] = (
        jnp.dot(zr, w_ref[...], preferred_element_type=jnp.float32) + b2_ref[...]
    )


_l2 = pl.pallas_call(
    _l2_body,
    grid=(NT,),
    in_specs=[
        pl.BlockSpec((T, H), lambda i: (i, 0)),
        pl.BlockSpec((1, H), lambda i: (0, 0)),
        pl.BlockSpec((1, H), lambda i: (0, 0)),
        pl.BlockSpec((1, H), lambda i: (0, 0)),
        pl.BlockSpec((1, H), lambda i: (0, 0)),
        pl.BlockSpec((H, H), lambda i: (0, 0)),
        pl.BlockSpec((1, H), lambda i: (0, 0)),
    ],
    out_specs=pl.BlockSpec((T, H), lambda i: (i, 0)),
    out_shape=jax.ShapeDtypeStruct((N, H), jnp.float32),
)


def _fin_body(c_ref, p0_ref, p1_ref, p2_ref, p3_ref, p4_ref, o_ref):
    inv = 1.0 / jnp.maximum(jnp.sum(c_ref[...], axis=0), 1.0)
    parts = [
        jnp.sum(p_ref[...], axis=0) * inv
        for p_ref in (p0_ref, p1_ref, p2_ref, p3_ref, p4_ref)
    ]
    o_ref[...] = jnp.concatenate(parts, axis=1)


_fin = pl.pallas_call(
    _fin_body,
    out_shape=jax.ShapeDtypeStruct((G, (L + 1) * H), jnp.float32),
)


def kernel(x, edge_index, batch, W_emb, b_emb, lin1_W, lin1_b, bn_g, bn_b, lin2_W, lin2_b):
    srcR = edge_index[0].reshape(E // CH, CH)
    dstR = edge_index[1].reshape(E // CH, CH)

    h = _embed(x, W_emb.T, b_emb.reshape(1, H))
    pools = []
    for l in range(L):
        h2 = h.reshape(NC * N, HH)
        agg = _agg(h2, srcR, dstR)
        pools.append(_pool(h2, batch).reshape(NC, G, H))
        z1, zs, zss = _l1(h, agg, lin1_W[l].T, lin1_b[l].reshape(1, H))
        h = _l2(z1, zs, zss, bn_g[l].reshape(1, H), bn_b[l].reshape(1, H),
                lin2_W[l].T, lin2_b[l].reshape(1, H))
    pool_last, cnt = _pool_cnt(h.reshape(NC * N, HH), batch)
    pools.append(pool_last.reshape(NC, G, H))
    return _fin(cnt.reshape(NC, G, H), *pools)
